# baseline (device time: 29053 ns/iter reference)
import jax
import jax.numpy as jnp
from jax import lax
from jax.experimental import pallas as pl
from jax.experimental.pallas import tpu as pltpu

T = 512
D = 1024
V_LOCAL = 8192
BLK = 1024
NBLK = V_LOCAL // BLK


def kernel(x, W, labels):
    def body(x_ref, w_ref, labels_ref, out_ref,
             logits_ref, acc_ref, comm_ref, send_sem, recv_sem):
        j = pl.program_id(0)
        my_x = lax.axis_index("x")
        my_y = lax.axis_index("y")
        partner = (1 - my_x, my_y)

        @pl.when(j == 0)
        def _():
            barrier_sem = pltpu.get_barrier_semaphore()
            pl.semaphore_signal(barrier_sem, inc=1, device_id=partner,
                                device_id_type=pl.DeviceIdType.MESH)
            pl.semaphore_wait(barrier_sem, 1)
            acc_ref[...] = jnp.zeros_like(acc_ref)

        logits_ref[j % 2] = jnp.dot(x_ref[...], w_ref[...],
                                    preferred_element_type=jnp.float32)

        def process(i):
            logits = logits_ref[i % 2]
            s_part = jnp.sum(jnp.exp(logits), axis=1, keepdims=True)
            local_label = labels_ref[...] - my_x * V_LOCAL - i * BLK
            col = lax.broadcasted_iota(jnp.int32, (T, BLK), 1)
            g_part = jnp.sum(jnp.where(col == local_label, logits, 0.0),
                             axis=1, keepdims=True)
            acc_ref[:, 0:1] += s_part
            acc_ref[:, 1:2] += g_part

        @pl.when(j > 0)
        def _():
            process(j - 1)

        @pl.when(j == NBLK - 1)
        def _():
            process(j)
            comm_ref[0, :, :] = acc_ref[...]
            rdma = pltpu.make_async_remote_copy(
                src_ref=comm_ref.at[0],
                dst_ref=comm_ref.at[1],
                send_sem=send_sem,
                recv_sem=recv_sem,
                device_id=partner,
                device_id_type=pl.DeviceIdType.MESH,
            )
            rdma.start()
            rdma.wait()
            s = acc_ref[:, 0:1] + comm_ref[1, :, 0:1]
            g = acc_ref[:, 1:2] + comm_ref[1, :, 1:2]
            out_ref[...] = jnp.log(s) - g

    out = pl.pallas_call(
        body,
        grid=(NBLK,),
        out_shape=jax.ShapeDtypeStruct((T, 1), jnp.float32),
        in_specs=[
            pl.BlockSpec((T, D), lambda j: (0, 0)),
            pl.BlockSpec((D, BLK), lambda j: (0, j)),
            pl.BlockSpec((T, 1), lambda j: (0, 0)),
        ],
        out_specs=pl.BlockSpec((T, 1), lambda j: (0, 0)),
        scratch_shapes=[
            pltpu.VMEM((2, T, BLK), jnp.float32),
            pltpu.VMEM((T, 8), jnp.float32),
            pltpu.VMEM((2, T, 8), jnp.float32),
            pltpu.SemaphoreType.DMA,
            pltpu.SemaphoreType.DMA,
        ],
        compiler_params=pltpu.CompilerParams(
            collective_id=0,
            dimension_semantics=("arbitrary",),
        ),
    )(x, W, labels.reshape(T, 1))

    return out.reshape(T)


# device time: 27451 ns/iter; 1.0584x vs baseline; 1.0584x over previous
import jax
import jax.numpy as jnp
from jax import lax
from jax.experimental import pallas as pl
from jax.experimental.pallas import tpu as pltpu

T = 512
D = 1024
V_LOCAL = 8192
BLK = 2048
NBLK = V_LOCAL // BLK


def kernel(x, W, labels):
    def body(x_ref, w_ref, labels_ref, out_ref,
             acc_ref, comm_ref, send_sem, recv_sem):
        j = pl.program_id(0)
        my_x = lax.axis_index("x")
        my_y = lax.axis_index("y")
        partner = (1 - my_x, my_y)

        @pl.when(j == 0)
        def _():
            barrier_sem = pltpu.get_barrier_semaphore()
            pl.semaphore_signal(barrier_sem, inc=1, device_id=partner,
                                device_id_type=pl.DeviceIdType.MESH)
            pl.semaphore_wait(barrier_sem, 1)
            acc_ref[...] = jnp.zeros_like(acc_ref)

        logits = jnp.dot(x_ref[...], w_ref[...],
                         preferred_element_type=jnp.float32)
        s_part = jnp.sum(jnp.exp(logits), axis=1, keepdims=True)
        local_label = labels_ref[...] - my_x * V_LOCAL - j * BLK
        col = lax.broadcasted_iota(jnp.int32, (T, BLK), 1)
        g_part = jnp.sum(jnp.where(col == local_label, logits, 0.0),
                         axis=1, keepdims=True)
        acc_ref[:, 0:1] += s_part
        acc_ref[:, 1:2] += g_part

        @pl.when(j == NBLK - 1)
        def _():
            comm_ref[0, :, :] = acc_ref[...]
            rdma = pltpu.make_async_remote_copy(
                src_ref=comm_ref.at[0],
                dst_ref=comm_ref.at[1],
                send_sem=send_sem,
                recv_sem=recv_sem,
                device_id=partner,
                device_id_type=pl.DeviceIdType.MESH,
            )
            rdma.start()
            rdma.wait()
            s = acc_ref[:, 0:1] + comm_ref[1, :, 0:1]
            g = acc_ref[:, 1:2] + comm_ref[1, :, 1:2]
            out_ref[...] = jnp.log(s) - g

    out = pl.pallas_call(
        body,
        grid=(NBLK,),
        out_shape=jax.ShapeDtypeStruct((T, 1), jnp.float32),
        in_specs=[
            pl.BlockSpec((T, D), lambda j: (0, 0)),
            pl.BlockSpec((D, BLK), lambda j: (0, j)),
            pl.BlockSpec((T, 1), lambda j: (0, 0)),
        ],
        out_specs=pl.BlockSpec((T, 1), lambda j: (0, 0)),
        scratch_shapes=[
            pltpu.VMEM((T, 8), jnp.float32),
            pltpu.VMEM((2, T, 8), jnp.float32),
            pltpu.SemaphoreType.DMA,
            pltpu.SemaphoreType.DMA,
        ],
        compiler_params=pltpu.CompilerParams(
            collective_id=0,
            dimension_semantics=("arbitrary",),
            vmem_limit_bytes=100 * 1024 * 1024,
        ),
    )(x, W, labels.reshape(T, 1))

    return out.reshape(T)


# device time: 26723 ns/iter; 1.0872x vs baseline; 1.0272x over previous
import jax
import jax.numpy as jnp
from jax import lax
from jax.experimental import pallas as pl
from jax.experimental.pallas import tpu as pltpu

T = 512
D = 1024
V_LOCAL = 8192
HALF = V_LOCAL // 2
CHK = 1024
NCHK = HALF // CHK


def kernel(x, W, labels):
    def body(x_ref, w_ref, labels_ref, out_ref,
             w_bufs, comm_ref, copy_sems, send_sems, recv_sems):
        my_x = lax.axis_index("x")
        my_y = lax.axis_index("y")
        peers = [
            (my_x, 1 - my_y),
            (1 - my_x, my_y),
            (1 - my_x, 1 - my_y),
        ]

        barrier_sem = pltpu.get_barrier_semaphore()
        for nbr in peers:
            pl.semaphore_signal(barrier_sem, inc=1, device_id=nbr,
                                device_id_type=pl.DeviceIdType.MESH)
        pl.semaphore_wait(barrier_sem, 3)

        base = my_y * HALF

        def wcopy(k):
            return pltpu.make_async_copy(
                w_ref.at[:, pl.ds(base + k * CHK, CHK)],
                w_bufs.at[k],
                copy_sems.at[k],
            )

        wcopy(0).start()

        s = jnp.zeros((T, 1), jnp.float32)
        g = jnp.zeros((T, 1), jnp.float32)
        col = lax.broadcasted_iota(jnp.int32, (T, CHK), 1)
        for k in range(NCHK):
            wcopy(k).wait()
            if k + 1 < NCHK:
                wcopy(k + 1).start()
            logits = jnp.dot(x_ref[...], w_bufs[k],
                             preferred_element_type=jnp.float32)
            s = s + jnp.sum(jnp.exp(logits), axis=1, keepdims=True)
            local_label = labels_ref[...] - my_x * V_LOCAL - base - k * CHK
            g = g + jnp.sum(jnp.where(col == local_label, logits, 0.0),
                            axis=1, keepdims=True)

        comm_ref[0, :, 0:1] = s
        comm_ref[0, :, 1:2] = g
        rdmas = []
        for p, nbr in enumerate(peers, start=1):
            r = pltpu.make_async_remote_copy(
                src_ref=comm_ref.at[0], dst_ref=comm_ref.at[p],
                send_sem=send_sems.at[p - 1], recv_sem=recv_sems.at[p - 1],
                device_id=nbr, device_id_type=pl.DeviceIdType.MESH)
            r.start()
            rdmas.append(r)
        for r in rdmas:
            r.wait()
        for p in range(1, 4):
            s = s + comm_ref[p, :, 0:1]
            g = g + comm_ref[p, :, 1:2]

        out_ref[...] = jnp.log(s) - g

    out = pl.pallas_call(
        body,
        out_shape=jax.ShapeDtypeStruct((T, 1), jnp.float32),
        in_specs=[
            pl.BlockSpec(memory_space=pltpu.VMEM),
            pl.BlockSpec(memory_space=pltpu.MemorySpace.HBM),
            pl.BlockSpec(memory_space=pltpu.VMEM),
        ],
        out_specs=pl.BlockSpec(memory_space=pltpu.VMEM),
        scratch_shapes=[
            pltpu.VMEM((NCHK, D, CHK), jnp.float32),
            pltpu.VMEM((4, T, 8), jnp.float32),
            pltpu.SemaphoreType.DMA((NCHK,)),
            pltpu.SemaphoreType.DMA((3,)),
            pltpu.SemaphoreType.DMA((3,)),
        ],
        compiler_params=pltpu.CompilerParams(
            collective_id=0,
            vmem_limit_bytes=64 * 1024 * 1024,
        ),
    )(x, W, labels.reshape(T, 1))

    return out.reshape(T)
